# baseline (speedup 1.0000x reference)
import functools

import jax
import jax.numpy as jnp
from jax import lax
from jax.experimental import pallas as pl
from jax.experimental.pallas import tpu as pltpu
from jax.experimental.pallas import tpu_sc as plsc

N = 10000
D = 128
H = 8
DH = 16
E = 320000

NC = 2
NS = 16
TILES = NC * NS
CHUNK = 128
EP = 327680
CPT = EP // (TILES * CHUNK)
EPT = EP // TILES

NT = 10240
RPS = NT // NS
NB = NT // 16
RPSB = NB // NS
DUMMY = N

_mesh = plsc.VectorSubcoreMesh(core_axis_name="c", subcore_axis_name="s")


CG = 64
RING = 5
NCH = EPT // CG
NGRP = NCH // RING
QW = D // 2


@jax.jit
def _sc_gather(kvt, qt, si, dig):
    @functools.partial(
        pl.kernel,
        mesh=_mesh,
        out_type=[
            jax.ShapeDtypeStruct((EP, D), jnp.float32),
            jax.ShapeDtypeStruct((EP, D), jnp.float32),
        ],
        scratch_types=[
            pltpu.VMEM((EPT,), jnp.int32),
            pltpu.VMEM((EPT,), jnp.int32),
        ] + [pltpu.VMEM((CG, D), jnp.float32) for _ in range(2 * RING)]
          + [pltpu.SemaphoreType.DMA for _ in range(2 * RING + 1)],
    )
    def k(kvt_hbm, qt_hbm, si_hbm, dig_hbm, kvj_hbm, qi_hbm,
          si_v, di_v, *bufs_and_sems):
        kv_b = bufs_and_sems[:RING]
        q_b = bufs_and_sems[RING:2 * RING]
        gs_kv = bufs_and_sems[2 * RING:3 * RING]
        gs_q = bufs_and_sems[3 * RING:4 * RING]
        ws = bufs_and_sems[4 * RING]
        wid = lax.axis_index("s") * NC + lax.axis_index("c")
        base = wid * EPT
        pltpu.sync_copy(si_hbm.at[pl.ds(base, EPT)], si_v)
        pltpu.sync_copy(dig_hbm.at[pl.ds(base, EPT)], di_v)

        @pl.loop(0, NGRP)
        def _(g):
            c0 = g * RING
            gh = []
            for b in range(RING):
                off = pl.multiple_of((c0 + b) * CG, CG)
                h1 = pltpu.async_copy(kvt_hbm.at[si_v.at[pl.ds(off, CG)]],
                                      kv_b[b], gs_kv[b])
                h2 = pltpu.async_copy(qt_hbm.at[di_v.at[pl.ds(off, CG)]],
                                      q_b[b], gs_q[b])
                gh.append((h1, h2))
            wh = []
            for b in range(RING):
                cb = pl.multiple_of(base + (c0 + b) * CG, CG)
                gh[b][0].wait()
                gh[b][1].wait()
                wh.append(pltpu.async_copy(kv_b[b], kvj_hbm.at[pl.ds(cb, CG)], ws))
                wh.append(pltpu.async_copy(q_b[b], qi_hbm.at[pl.ds(cb, CG)], ws))
            for h in wh:
                h.wait()

    return k(kvt, qt, si, dig)


@jax.jit
def _sc_scatter(msga, msgb, disa, disb, zeros):
    @functools.partial(
        pl.kernel,
        mesh=_mesh,
        out_type=[
            jax.ShapeDtypeStruct((NC, NT, D), jnp.float32),
            jax.ShapeDtypeStruct((NC, NB, D), jnp.float32),
        ],
        scratch_types=[
            pltpu.VMEM((CHUNK,), jnp.int32),
            pltpu.VMEM((CHUNK,), jnp.int32),
            pltpu.VMEM((CHUNK, D), jnp.float32),
            pltpu.VMEM((CHUNK, D), jnp.float32),
            pltpu.VMEM_SHARED((NT, D), jnp.float32),
            pltpu.VMEM_SHARED((NB, D), jnp.float32),
        ],
    )
    def k(msga_hbm, msgb_hbm, disa_hbm, disb_hbm, zeros_hbm, outa_hbm,
          outb_hbm, ia_v, ib_v, ma_v, mb_v, acca_sh, accb_sh):
        c = lax.axis_index("c")
        s = lax.axis_index("s")
        wid = s * NC + c
        base = wid * EPT
        ra = pl.multiple_of(s * RPS, 8)
        rb = pl.multiple_of(s * RPSB, 8)

        pltpu.sync_copy(zeros_hbm.at[pl.ds(ra, RPS)], acca_sh.at[pl.ds(ra, RPS)])
        pltpu.sync_copy(zeros_hbm.at[pl.ds(rb, RPSB)], accb_sh.at[pl.ds(rb, RPSB)])
        plsc.subcore_barrier()

        @pl.loop(0, CPT)
        def _(j):
            cb = pl.multiple_of(base + j * CHUNK, CHUNK)
            pltpu.sync_copy(disa_hbm.at[pl.ds(cb, CHUNK)], ia_v)
            pltpu.sync_copy(disb_hbm.at[pl.ds(cb, CHUNK)], ib_v)
            pltpu.sync_copy(msga_hbm.at[pl.ds(cb, CHUNK)], ma_v)
            pltpu.sync_copy(msgb_hbm.at[pl.ds(cb, CHUNK)], mb_v)
            pltpu.sync_copy(ma_v, acca_sh.at[ia_v], add=True)
            pltpu.sync_copy(mb_v, accb_sh.at[ib_v], add=True)

        plsc.subcore_barrier()
        pltpu.sync_copy(acca_sh.at[pl.ds(ra, RPS)], outa_hbm.at[c, pl.ds(ra, RPS)])
        pltpu.sync_copy(accb_sh.at[pl.ds(rb, RPSB)], outb_hbm.at[c, pl.ds(rb, RPSB)])

    return k(msga, msgb, disa, disb, zeros)


def _mm(x, w, b, act=None, out_dtype=jnp.float32):
    n, din = x.shape
    cout = w.shape[1]
    blk = 2000 if n % 2000 == 0 else 2048

    def body(x_ref, w_ref, b_ref, o_ref):
        y = jnp.dot(x_ref[...], w_ref[...],
                    preferred_element_type=jnp.float32) + b_ref[...]
        if act == "relu":
            y = jnp.maximum(y, 0.0)
        o_ref[...] = y.astype(out_dtype)

    return pl.pallas_call(
        body,
        grid=(n // blk,),
        in_specs=[
            pl.BlockSpec((blk, din), lambda i: (i, 0)),
            pl.BlockSpec((din, cout), lambda i: (0, 0)),
            pl.BlockSpec((1, cout), lambda i: (0, 0)),
        ],
        out_specs=pl.BlockSpec((blk, cout), lambda i: (i, 0)),
        out_shape=jax.ShapeDtypeStruct((n, cout), out_dtype),
    )(x, w, b.reshape(1, cout))


def _pack_bf16(x_bf):
    n, m = x_bf.shape
    return lax.bitcast_convert_type(x_bf.reshape(n, m // 2, 2), jnp.float32)


_P1_BLK = 2048


def _unpack_bf16_pair(w):
    wi = lax.bitcast_convert_type(w, jnp.int32)
    lo = lax.bitcast_convert_type(wi << 16, jnp.float32)
    hi = lax.bitcast_convert_type(wi & jnp.int32(-65536), jnp.float32)
    return lo, hi


def _p1_body(kvj_ref, qi_ref, lg_ref, gm_ref):
    lo_k, hi_k = _unpack_bf16_pair(kvj_ref[...][:, :QW])
    q = qi_ref[...]
    prod = lo_k * q[:, :QW] + hi_k * q[:, QW:]
    lg = jnp.sum(prod.reshape(_P1_BLK, H, DH // 2), axis=-1)
    lg_ref[...] = lg
    prev = jnp.where(pl.program_id(0) == 0, -1e30, gm_ref[0, 0])
    gm_ref[...] = jnp.maximum(prev, jnp.max(lg)).reshape(1, 1)


def _p1(kvj, qi):
    return pl.pallas_call(
        _p1_body,
        grid=(EP // _P1_BLK,),
        in_specs=[
            pl.BlockSpec((_P1_BLK, D), lambda i: (i, 0)),
            pl.BlockSpec((_P1_BLK, D), lambda i: (i, 0)),
        ],
        out_specs=[
            pl.BlockSpec((_P1_BLK, H), lambda i: (i, 0)),
            pl.BlockSpec((1, 1), lambda i: (0, 0)),
        ],
        out_shape=[
            jax.ShapeDtypeStruct((EP, H), jnp.float32),
            jax.ShapeDtypeStruct((1, 1), jnp.float32),
        ],
    )(kvj, qi)


def _p2_body(lg_ref, gm_ref, kvj_ref, dis_ref, msga_ref, msgb_ref):
    ex = jnp.exp(lg_ref[...] - gm_ref[0, 0])
    lo_v, hi_v = _unpack_bf16_pair(kvj_ref[...][:, QW:])
    exb = jnp.broadcast_to(ex[:, :, None],
                           (_P1_BLK, H, DH // 2)).reshape(_P1_BLK, QW)
    msga_ref[...] = jnp.concatenate([exb * lo_v, exb * hi_v], axis=1)
    ex16 = jnp.broadcast_to(ex[:, None, :], (_P1_BLK, 16, H)).reshape(_P1_BLK, D)
    slot = lax.broadcasted_iota(jnp.int32, (_P1_BLK, D), 1) // H
    m = slot == (dis_ref[...] & 15)
    msgb_ref[...] = jnp.where(m, ex16, 0.0)


def _p2(lg, gm, vj, dis2):
    return pl.pallas_call(
        _p2_body,
        grid=(EP // _P1_BLK,),
        in_specs=[
            pl.BlockSpec((_P1_BLK, H), lambda i: (i, 0)),
            pl.BlockSpec((1, 1), lambda i: (0, 0)),
            pl.BlockSpec((_P1_BLK, D), lambda i: (i, 0)),
            pl.BlockSpec((_P1_BLK, 1), lambda i: (i, 0)),
        ],
        out_specs=[
            pl.BlockSpec((_P1_BLK, D), lambda i: (i, 0)),
            pl.BlockSpec((_P1_BLK, D), lambda i: (i, 0)),
        ],
        out_shape=[
            jax.ShapeDtypeStruct((EP, D), jnp.float32),
            jax.ShapeDtypeStruct((EP, D), jnp.float32),
        ],
    )(lg, gm, vj, dis2)


_EPI_BLK = 2000


def _epi_body(acc_ref, s_ref, x_ref, wa_ref, ba_ref, sk_ref, o_ref):
    num = acc_ref[0] + acc_ref[1]
    s8 = s_ref[0] + s_ref[1]
    d64 = jnp.broadcast_to(s8[:, :, None],
                           (_EPI_BLK, H, DH // 2)).reshape(_EPI_BLK, QW)
    den = jnp.concatenate([d64, d64], axis=1)
    o = num / (den + 1e-16)
    g = jax.nn.gelu(o)
    y = jnp.dot(g, wa_ref[...], preferred_element_type=jnp.float32) + ba_ref[...]
    sa = jax.nn.sigmoid(sk_ref[0, 0])
    o_ref[...] = sa * y + (1.0 - sa) * x_ref[...]


def _epilogue(acc2, s2, x, wa, ba, skip):
    return pl.pallas_call(
        _epi_body,
        grid=(N // _EPI_BLK,),
        in_specs=[
            pl.BlockSpec((2, _EPI_BLK, D), lambda i: (0, i, 0)),
            pl.BlockSpec((2, _EPI_BLK, H), lambda i: (0, i, 0)),
            pl.BlockSpec((_EPI_BLK, D), lambda i: (i, 0)),
            pl.BlockSpec((D, D), lambda i: (0, 0)),
            pl.BlockSpec((1, D), lambda i: (0, 0)),
            pl.BlockSpec((1, 1), lambda i: (0, 0)),
        ],
        out_specs=pl.BlockSpec((_EPI_BLK, D), lambda i: (i, 0)),
        out_shape=jax.ShapeDtypeStruct((N, D), jnp.float32),
    )(acc2, s2, x, wa, ba.reshape(1, D), skip.reshape(1, 1))


def _fold(lin, A):
    w = jnp.einsum("ihd,hde->ihe", lin["W"].reshape(D, H, DH), A).reshape(D, D)
    b = jnp.einsum("hd,hde->he", lin["b"].reshape(H, DH), A).reshape(D)
    return w, b


def _scale_q(lin, p_rel):
    s = p_rel / jnp.sqrt(jnp.float32(DH))
    w = (lin["W"].reshape(D, H, DH) * s[None, :, None]).reshape(D, D)
    b = (lin["b"].reshape(H, DH) * s[:, None]).reshape(D)
    return w, b


def _round2(ra, rb, zeros):
    out = []
    g = [_sc_gather(kv_tab, q_tab, si, dig)
         for (kv_tab, q_tab, si, dig, dis2, disb) in (ra, rb)]
    p1 = [_p1(kvj, qi) for (kvj, qi) in g]
    p2 = [_p2(p1[i][0], p1[i][1], g[i][0], (ra, rb)[i][4]) for i in (0, 1)]
    for i, (kv_tab, q_tab, si, dig, dis2, disb) in enumerate((ra, rb)):
        acca, accb = _sc_scatter(p2[i][0], p2[i][1], dis2.reshape(EP),
                                 disb, zeros)
        out.append((acca, accb.reshape(NC, NT, H)))
    return out


def kernel(x_user, x_location, edge_index_user_location,
           edge_index_location_user, params):
    pad = EP - E
    si_ul = jnp.pad(edge_index_user_location[0], (0, pad))
    dig_ul = jnp.pad(edge_index_user_location[1], (0, pad))
    dis_ul = jnp.pad(edge_index_user_location[1], (0, pad),
                     constant_values=DUMMY)
    si_lu = jnp.pad(edge_index_location_user[0], (0, pad))
    dig_lu = jnp.pad(edge_index_location_user[1], (0, pad))
    dis_lu = jnp.pad(edge_index_location_user[1], (0, pad),
                     constant_values=DUMMY)
    dis2_ul = dis_ul.reshape(EP, 1)
    dis2_lu = dis_lu.reshape(EP, 1)
    disb_ul = dis_ul >> 4
    disb_lu = dis_lu >> 4
    zeros = jnp.zeros((NT, D), jnp.float32)
    perm = jnp.concatenate([jnp.arange(0, D, 2), jnp.arange(1, D, 2)])

    p = params
    x_u = _mm(x_user, p["lin_dict"]["user"]["W"], p["lin_dict"]["user"]["b"],
              act="relu")
    x_l = _mm(x_location, p["lin_dict"]["location"]["W"],
              p["lin_dict"]["location"]["b"], act="relu")

    for lp in p["convs"]:
        et1 = "user__visits__location"
        et2 = "location__rev_visits__user"
        wk1, bk1 = _fold(lp["k"]["user"], lp["a_rel"][et1])
        wv1, bv1 = _fold(lp["v"]["user"], lp["m_rel"][et1])
        wq1, bq1 = _scale_q(lp["q"]["location"], lp["p_rel"][et1])
        wk2, bk2 = _fold(lp["k"]["location"], lp["a_rel"][et2])
        wv2, bv2 = _fold(lp["v"]["location"], lp["m_rel"][et2])
        wq2, bq2 = _scale_q(lp["q"]["user"], lp["p_rel"][et2])

        kv_u = _pack_bf16(_mm(x_u, jnp.concatenate([wk1, wv1], axis=1),
                              jnp.concatenate([bk1, bv1]),
                              out_dtype=jnp.bfloat16))
        q_l = _mm(x_l, wq1[:, perm], bq1[perm])
        kv_l = _pack_bf16(_mm(x_l, jnp.concatenate([wk2, wv2], axis=1),
                              jnp.concatenate([bk2, bv2]),
                              out_dtype=jnp.bfloat16))
        q_u = _mm(x_u, wq2[:, perm], bq2[perm])

        (acc_l, s_l), (acc_u, s_u) = _round2(
            (kv_u, q_l, si_ul, dig_ul, dis2_ul, disb_ul),
            (kv_l, q_u, si_lu, dig_lu, dis2_lu, disb_lu), zeros)

        x_l_new = _epilogue(acc_l[:, :N, :], s_l[:, :N, :], x_l,
                            lp["a"]["location"]["W"][perm, :],
                            lp["a"]["location"]["b"], lp["skip"]["location"])
        x_u_new = _epilogue(acc_u[:, :N, :], s_u[:, :N, :], x_u,
                            lp["a"]["user"]["W"][perm, :],
                            lp["a"]["user"]["b"], lp["skip"]["user"])
        x_u, x_l = x_u_new, x_l_new

    u = _mm(x_u, p["lin1"]["W"], p["lin1"]["b"])
    l = _mm(x_l, p["lin2"]["W"], p["lin2"]["b"])
    return (u, l)


# speedup vs baseline: 1.9816x; 1.9816x over previous
import functools

import jax
import jax.numpy as jnp
from jax import lax
from jax.experimental import pallas as pl
from jax.experimental.pallas import tpu as pltpu
from jax.experimental.pallas import tpu_sc as plsc

N = 10000
D = 128
H = 8
DH = 16
E = 320000

NC = 2
NS = 16
TILES = NC * NS
CHUNK = 128
EP = 327680
CPT = EP // (TILES * CHUNK)
EPT = EP // TILES

NT = 10240
RPS = NT // NS
NB = NT // 16
RPSB = NB // NS
DUMMY = N

_mesh = plsc.VectorSubcoreMesh(core_axis_name="c", subcore_axis_name="s")


CG = 64
RING = 5
NCH = EPT // CG
NGRP = NCH // RING
QW = D // 2


@jax.jit
def _sc_gather(kvt, qt, si, dig):
    @functools.partial(
        pl.kernel,
        mesh=_mesh,
        out_type=[
            jax.ShapeDtypeStruct((EP, D), jnp.float32),
            jax.ShapeDtypeStruct((EP, D), jnp.float32),
        ],
        scratch_types=[
            pltpu.VMEM((EPT,), jnp.int32),
            pltpu.VMEM((EPT,), jnp.int32),
        ] + [pltpu.VMEM((CG, D), jnp.float32) for _ in range(2 * RING)]
          + [pltpu.SemaphoreType.DMA for _ in range(2 * RING + 1)],
    )
    def k(kvt_hbm, qt_hbm, si_hbm, dig_hbm, kvj_hbm, qi_hbm,
          si_v, di_v, *bufs_and_sems):
        kv_b = bufs_and_sems[:RING]
        q_b = bufs_and_sems[RING:2 * RING]
        gs_kv = bufs_and_sems[2 * RING:3 * RING]
        gs_q = bufs_and_sems[3 * RING:4 * RING]
        ws = bufs_and_sems[4 * RING]
        wid = lax.axis_index("s") * NC + lax.axis_index("c")
        base = wid * EPT
        pltpu.sync_copy(si_hbm.at[pl.ds(base, EPT)], si_v)
        pltpu.sync_copy(dig_hbm.at[pl.ds(base, EPT)], di_v)

        @pl.loop(0, NGRP)
        def _(g):
            c0 = g * RING
            gh = []
            for b in range(RING):
                off = pl.multiple_of((c0 + b) * CG, CG)
                h1 = pltpu.async_copy(kvt_hbm.at[si_v.at[pl.ds(off, CG)]],
                                      kv_b[b], gs_kv[b])
                h2 = pltpu.async_copy(qt_hbm.at[di_v.at[pl.ds(off, CG)]],
                                      q_b[b], gs_q[b])
                gh.append((h1, h2))
            wh = []
            for b in range(RING):
                cb = pl.multiple_of(base + (c0 + b) * CG, CG)
                gh[b][0].wait()
                gh[b][1].wait()
                wh.append(pltpu.async_copy(kv_b[b], kvj_hbm.at[pl.ds(cb, CG)], ws))
                wh.append(pltpu.async_copy(q_b[b], qi_hbm.at[pl.ds(cb, CG)], ws))
            for h in wh:
                h.wait()

    return k(kvt, qt, si, dig)


@jax.jit
def _sc_scatter(msga, msgb, disa, disb, zeros):
    @functools.partial(
        pl.kernel,
        mesh=_mesh,
        out_type=[
            jax.ShapeDtypeStruct((NC, NT, D), jnp.float32),
            jax.ShapeDtypeStruct((NC, NB, D), jnp.float32),
        ],
        scratch_types=[
            pltpu.VMEM((CHUNK,), jnp.int32),
            pltpu.VMEM((CHUNK,), jnp.int32),
            pltpu.VMEM((CHUNK, D), jnp.float32),
            pltpu.VMEM((CHUNK, D), jnp.float32),
            pltpu.VMEM_SHARED((NT, D), jnp.float32),
            pltpu.VMEM_SHARED((NB, D), jnp.float32),
        ],
    )
    def k(msga_hbm, msgb_hbm, disa_hbm, disb_hbm, zeros_hbm, outa_hbm,
          outb_hbm, ia_v, ib_v, ma_v, mb_v, acca_sh, accb_sh):
        c = lax.axis_index("c")
        s = lax.axis_index("s")
        wid = s * NC + c
        base = wid * EPT
        ra = pl.multiple_of(s * RPS, 8)
        rb = pl.multiple_of(s * RPSB, 8)

        pltpu.sync_copy(zeros_hbm.at[pl.ds(ra, RPS)], acca_sh.at[pl.ds(ra, RPS)])
        pltpu.sync_copy(zeros_hbm.at[pl.ds(rb, RPSB)], accb_sh.at[pl.ds(rb, RPSB)])
        plsc.subcore_barrier()

        @pl.loop(0, CPT)
        def _(j):
            cb = pl.multiple_of(base + j * CHUNK, CHUNK)
            pltpu.sync_copy(disa_hbm.at[pl.ds(cb, CHUNK)], ia_v)
            pltpu.sync_copy(disb_hbm.at[pl.ds(cb, CHUNK)], ib_v)
            pltpu.sync_copy(msga_hbm.at[pl.ds(cb, CHUNK)], ma_v)
            pltpu.sync_copy(msgb_hbm.at[pl.ds(cb, CHUNK)], mb_v)
            pltpu.sync_copy(ma_v, acca_sh.at[ia_v], add=True)
            pltpu.sync_copy(mb_v, accb_sh.at[ib_v], add=True)

        plsc.subcore_barrier()
        pltpu.sync_copy(acca_sh.at[pl.ds(ra, RPS)], outa_hbm.at[c, pl.ds(ra, RPS)])
        pltpu.sync_copy(accb_sh.at[pl.ds(rb, RPSB)], outb_hbm.at[c, pl.ds(rb, RPSB)])

    return k(msga, msgb, disa, disb, zeros)


def _mm(x, w, b, act=None, out_dtype=jnp.float32):
    n, din = x.shape
    cout = w.shape[1]
    blk = 2000 if n % 2000 == 0 else 2048

    def body(x_ref, w_ref, b_ref, o_ref):
        y = jnp.dot(x_ref[...], w_ref[...],
                    preferred_element_type=jnp.float32) + b_ref[...]
        if act == "relu":
            y = jnp.maximum(y, 0.0)
        o_ref[...] = y.astype(out_dtype)

    return pl.pallas_call(
        body,
        grid=(n // blk,),
        in_specs=[
            pl.BlockSpec((blk, din), lambda i: (i, 0)),
            pl.BlockSpec((din, cout), lambda i: (0, 0)),
            pl.BlockSpec((1, cout), lambda i: (0, 0)),
        ],
        out_specs=pl.BlockSpec((blk, cout), lambda i: (i, 0)),
        out_shape=jax.ShapeDtypeStruct((n, cout), out_dtype),
    )(x, w, b.reshape(1, cout))


def _pack_bf16(x_bf):
    n, m = x_bf.shape
    return lax.bitcast_convert_type(x_bf.reshape(n, m // 2, 2), jnp.float32)


_P1_BLK = 2048


def _unpack_bf16_pair(w):
    wi = lax.bitcast_convert_type(w, jnp.int32)
    lo = lax.bitcast_convert_type(wi << 16, jnp.float32)
    hi = lax.bitcast_convert_type(wi & jnp.int32(-65536), jnp.float32)
    return lo, hi


def _hmm(a, b):
    return lax.dot_general(a, b, (((1,), (0,)), ((), ())),
                           precision=lax.Precision.HIGHEST,
                           preferred_element_type=jnp.float32)


def _p1_body(kvj_ref, qi_ref, sum_ref, lg_ref, gm_ref):
    lo_k, hi_k = _unpack_bf16_pair(kvj_ref[...][:, :QW])
    q = qi_ref[...]
    prod = lo_k * q[:, :QW] + hi_k * q[:, QW:]
    lg = _hmm(prod, sum_ref[...])
    lg_ref[...] = lg
    prev = jnp.where(pl.program_id(0) == 0, -1e30, gm_ref[0, 0])
    gm_ref[...] = jnp.maximum(prev, jnp.max(lg)).reshape(1, 1)


def _p1(kvj, qi, selsum):
    return pl.pallas_call(
        _p1_body,
        grid=(EP // _P1_BLK,),
        in_specs=[
            pl.BlockSpec((_P1_BLK, D), lambda i: (i, 0)),
            pl.BlockSpec((_P1_BLK, D), lambda i: (i, 0)),
            pl.BlockSpec((QW, H), lambda i: (0, 0)),
        ],
        out_specs=[
            pl.BlockSpec((_P1_BLK, H), lambda i: (i, 0)),
            pl.BlockSpec((1, 1), lambda i: (0, 0)),
        ],
        out_shape=[
            jax.ShapeDtypeStruct((EP, H), jnp.float32),
            jax.ShapeDtypeStruct((1, 1), jnp.float32),
        ],
    )(kvj, qi, selsum)


def _p2_body(lg_ref, gm_ref, kvj_ref, dis_ref, s64_ref, s128_ref,
             msga_ref, msgb_ref):
    ex = jnp.exp(lg_ref[...] - gm_ref[0, 0])
    lo_v, hi_v = _unpack_bf16_pair(kvj_ref[...][:, QW:])
    exb = _hmm(ex, s64_ref[...])
    msga_ref[...] = jnp.concatenate([exb * lo_v, exb * hi_v], axis=1)
    ex16 = _hmm(ex, s128_ref[...])
    slot = lax.broadcasted_iota(jnp.int32, (_P1_BLK, D), 1) // H
    m = slot == (dis_ref[...] & 15)
    msgb_ref[...] = jnp.where(m, ex16, 0.0)


def _p2(lg, gm, vj, dis2, sel64, sel128):
    return pl.pallas_call(
        _p2_body,
        grid=(EP // _P1_BLK,),
        in_specs=[
            pl.BlockSpec((_P1_BLK, H), lambda i: (i, 0)),
            pl.BlockSpec((1, 1), lambda i: (0, 0)),
            pl.BlockSpec((_P1_BLK, D), lambda i: (i, 0)),
            pl.BlockSpec((_P1_BLK, 1), lambda i: (i, 0)),
            pl.BlockSpec((H, QW), lambda i: (0, 0)),
            pl.BlockSpec((H, D), lambda i: (0, 0)),
        ],
        out_specs=[
            pl.BlockSpec((_P1_BLK, D), lambda i: (i, 0)),
            pl.BlockSpec((_P1_BLK, D), lambda i: (i, 0)),
        ],
        out_shape=[
            jax.ShapeDtypeStruct((EP, D), jnp.float32),
            jax.ShapeDtypeStruct((EP, D), jnp.float32),
        ],
    )(lg, gm, vj, dis2, sel64, sel128)


_EPI_BLK = 2000


def _epi_body(acc_ref, s_ref, x_ref, wa_ref, ba_ref, sk_ref, o_ref):
    num = acc_ref[0] + acc_ref[1]
    s8 = s_ref[0] + s_ref[1]
    d64 = jnp.broadcast_to(s8[:, :, None],
                           (_EPI_BLK, H, DH // 2)).reshape(_EPI_BLK, QW)
    den = jnp.concatenate([d64, d64], axis=1)
    o = num / (den + 1e-16)
    g = jax.nn.gelu(o)
    y = jnp.dot(g, wa_ref[...], preferred_element_type=jnp.float32) + ba_ref[...]
    sa = jax.nn.sigmoid(sk_ref[0, 0])
    o_ref[...] = sa * y + (1.0 - sa) * x_ref[...]


def _epilogue(acc2, s2, x, wa, ba, skip):
    return pl.pallas_call(
        _epi_body,
        grid=(N // _EPI_BLK,),
        in_specs=[
            pl.BlockSpec((2, _EPI_BLK, D), lambda i: (0, i, 0)),
            pl.BlockSpec((2, _EPI_BLK, H), lambda i: (0, i, 0)),
            pl.BlockSpec((_EPI_BLK, D), lambda i: (i, 0)),
            pl.BlockSpec((D, D), lambda i: (0, 0)),
            pl.BlockSpec((1, D), lambda i: (0, 0)),
            pl.BlockSpec((1, 1), lambda i: (0, 0)),
        ],
        out_specs=pl.BlockSpec((_EPI_BLK, D), lambda i: (i, 0)),
        out_shape=jax.ShapeDtypeStruct((N, D), jnp.float32),
    )(acc2, s2, x, wa, ba.reshape(1, D), skip.reshape(1, 1))


def _fold(lin, A):
    w = jnp.einsum("ihd,hde->ihe", lin["W"].reshape(D, H, DH), A).reshape(D, D)
    b = jnp.einsum("hd,hde->he", lin["b"].reshape(H, DH), A).reshape(D)
    return w, b


def _scale_q(lin, p_rel):
    s = p_rel / jnp.sqrt(jnp.float32(DH))
    w = (lin["W"].reshape(D, H, DH) * s[None, :, None]).reshape(D, D)
    b = (lin["b"].reshape(H, DH) * s[:, None]).reshape(D)
    return w, b


def _round2(ra, rb, zeros):
    lane64 = jnp.arange(QW) // (DH // 2)
    lane128 = jnp.arange(D) % H
    selsum = (lane64[:, None] == jnp.arange(H)[None, :]).astype(jnp.float32)
    sel64 = (jnp.arange(H)[:, None] == lane64[None, :]).astype(jnp.float32)
    sel128 = (jnp.arange(H)[:, None] == lane128[None, :]).astype(jnp.float32)
    out = []
    g = [_sc_gather(kv_tab, q_tab, si, dig)
         for (kv_tab, q_tab, si, dig, dis2, disb) in (ra, rb)]
    p1 = [_p1(kvj, qi, selsum) for (kvj, qi) in g]
    p2 = [_p2(p1[i][0], p1[i][1], g[i][0], (ra, rb)[i][4], sel64, sel128)
          for i in (0, 1)]
    for i, (kv_tab, q_tab, si, dig, dis2, disb) in enumerate((ra, rb)):
        acca, accb = _sc_scatter(p2[i][0], p2[i][1], dis2.reshape(EP),
                                 disb, zeros)
        out.append((acca, accb.reshape(NC, NT, H)))
    return out


def kernel(x_user, x_location, edge_index_user_location,
           edge_index_location_user, params):
    pad = EP - E
    si_ul = jnp.pad(edge_index_user_location[0], (0, pad))
    dig_ul = jnp.pad(edge_index_user_location[1], (0, pad))
    dis_ul = jnp.pad(edge_index_user_location[1], (0, pad),
                     constant_values=DUMMY)
    si_lu = jnp.pad(edge_index_location_user[0], (0, pad))
    dig_lu = jnp.pad(edge_index_location_user[1], (0, pad))
    dis_lu = jnp.pad(edge_index_location_user[1], (0, pad),
                     constant_values=DUMMY)
    dis2_ul = dis_ul.reshape(EP, 1)
    dis2_lu = dis_lu.reshape(EP, 1)
    disb_ul = dis_ul >> 4
    disb_lu = dis_lu >> 4
    zeros = jnp.zeros((NT, D), jnp.float32)
    perm = jnp.concatenate([jnp.arange(0, D, 2), jnp.arange(1, D, 2)])

    p = params
    x_u = _mm(x_user, p["lin_dict"]["user"]["W"], p["lin_dict"]["user"]["b"],
              act="relu")
    x_l = _mm(x_location, p["lin_dict"]["location"]["W"],
              p["lin_dict"]["location"]["b"], act="relu")

    for lp in p["convs"]:
        et1 = "user__visits__location"
        et2 = "location__rev_visits__user"
        wk1, bk1 = _fold(lp["k"]["user"], lp["a_rel"][et1])
        wv1, bv1 = _fold(lp["v"]["user"], lp["m_rel"][et1])
        wq1, bq1 = _scale_q(lp["q"]["location"], lp["p_rel"][et1])
        wk2, bk2 = _fold(lp["k"]["location"], lp["a_rel"][et2])
        wv2, bv2 = _fold(lp["v"]["location"], lp["m_rel"][et2])
        wq2, bq2 = _scale_q(lp["q"]["user"], lp["p_rel"][et2])

        kv_u = _pack_bf16(_mm(x_u, jnp.concatenate([wk1, wv1], axis=1),
                              jnp.concatenate([bk1, bv1]),
                              out_dtype=jnp.bfloat16))
        q_l = _mm(x_l, wq1[:, perm], bq1[perm])
        kv_l = _pack_bf16(_mm(x_l, jnp.concatenate([wk2, wv2], axis=1),
                              jnp.concatenate([bk2, bv2]),
                              out_dtype=jnp.bfloat16))
        q_u = _mm(x_u, wq2[:, perm], bq2[perm])

        (acc_l, s_l), (acc_u, s_u) = _round2(
            (kv_u, q_l, si_ul, dig_ul, dis2_ul, disb_ul),
            (kv_l, q_u, si_lu, dig_lu, dis2_lu, disb_lu), zeros)

        x_l_new = _epilogue(acc_l[:, :N, :], s_l[:, :N, :], x_l,
                            lp["a"]["location"]["W"][perm, :],
                            lp["a"]["location"]["b"], lp["skip"]["location"])
        x_u_new = _epilogue(acc_u[:, :N, :], s_u[:, :N, :], x_u,
                            lp["a"]["user"]["W"][perm, :],
                            lp["a"]["user"]["b"], lp["skip"]["user"])
        x_u, x_l = x_u_new, x_l_new

    u = _mm(x_u, p["lin1"]["W"], p["lin1"]["b"])
    l = _mm(x_l, p["lin2"]["W"], p["lin2"]["b"])
    return (u, l)
